# baseline (speedup 1.0000x reference)
import functools

import jax
import jax.numpy as jnp
from jax import lax
from jax.experimental import pallas as pl
from jax.experimental.pallas import tpu as pltpu
from jax.experimental.pallas import tpu_sc as plsc

C = 21
ROW = 32
NROWS = C + 1
HW = ROW * NROWS
DEAD = C * ROW

L = 16
NSC = 2
NSUB = 16
NW = NSC * NSUB


def _argmax_body(x_ref, t_ref, o_ref):
    x = x_ref[0]
    t = t_ref[0]
    m = x[0]
    seg = jnp.zeros(m.shape, jnp.int32)
    for c in range(1, C):
        xc = x[c]
        gt = xc > m
        m = jnp.where(gt, xc, m)
        seg = jnp.where(gt, c, seg)
    valid = (t >= 0) & (t < C)
    idx = jnp.where(valid, t * ROW + seg, DEAD)
    o_ref[...] = idx.reshape(o_ref.shape)


def _argmax_call(x4, t3, rows):
    b, _, h, w = x4.shape
    nb = h // rows
    return pl.pallas_call(
        _argmax_body,
        grid=(b, nb),
        in_specs=[
            pl.BlockSpec((1, C, rows, w), lambda i, j: (i, 0, j, 0)),
            pl.BlockSpec((1, rows, w), lambda i, j: (i, j, 0)),
        ],
        out_specs=pl.BlockSpec((rows * w,), lambda i, j, nb=nb: (i * nb + j,)),
        out_shape=jax.ShapeDtypeStruct((b * h * w,), jnp.int32),
    )(x4, t3)


HSTRIDE = HW + 1


def _sc_hist_body(idx_hbm, part_hbm, idx_v, hist_v, fold_v, *, per_w):
    wid = lax.axis_index("s") * NSC + lax.axis_index("c")
    pltpu.sync_copy(idx_hbm.at[pl.ds(wid * per_w, per_w)], idx_v)

    zeros = jnp.zeros((L,), jnp.float32)
    lane_iota = lax.iota(jnp.int32, L)
    for i in range(L * HSTRIDE // L + 1):
        plsc.store_scatter(hist_v, [jnp.minimum(i * L + lane_iota,
                                                L * HSTRIDE - 1)], zeros)

    lane_base = lane_iota * HSTRIDE
    ones = jnp.full((L,), 1.0, jnp.float32)

    @plsc.parallel_loop(0, per_w, L, unroll=16)
    def _(i):
        v = idx_v[pl.ds(i, L)]
        plsc.addupdate_scatter(hist_v, [lane_base + v], ones)

    for c in range(HW // L):
        acc = zeros
        for r in range(L):
            acc = acc + plsc.load_gather(hist_v, [r * HSTRIDE + c * L + lane_iota])
        fold_v[pl.ds(c * L, L)] = acc

    pltpu.sync_copy(fold_v, part_hbm.at[wid])


def _sc_hist_call(idx_flat):
    n = idx_flat.shape[0]
    per_w = n // NW
    mesh = plsc.VectorSubcoreMesh(core_axis_name="c", subcore_axis_name="s")
    return pl.kernel(
        functools.partial(_sc_hist_body, per_w=per_w),
        out_type=jax.ShapeDtypeStruct((NW, HW), jnp.float32),
        mesh=mesh,
        compiler_params=pltpu.CompilerParams(needs_layout_passes=False),
        scratch_types=[
            pltpu.VMEM((per_w,), jnp.int32),
            pltpu.VMEM((L * HSTRIDE,), jnp.float32),
            pltpu.VMEM((HW,), jnp.float32),
        ],
    )(idx_flat)


def _dice_body(p_ref, o_ref):
    m = p_ref[...]
    t_io = lax.broadcasted_iota(jnp.int32, (NROWS, NW * NROWS), 0)
    r_io = lax.broadcasted_iota(jnp.int32, (NROWS, NW * NROWS), 1)
    sel = (r_io % NROWS == t_io).astype(jnp.float32)
    mat22 = lax.dot_general(sel, m, (((1,), (0,)), ((), ())),
                            preferred_element_type=jnp.float32)
    mat = mat22[0:C, :]
    eye = (lax.broadcasted_iota(jnp.int32, (C, ROW), 0)
           == lax.broadcasted_iota(jnp.int32, (C, ROW), 1))
    tp = jnp.sum(jnp.where(eye, mat, 0.0), axis=1, keepdims=True)
    fp_all = jnp.sum(mat, axis=1, keepdims=True)
    ones_c = jnp.full((C, 1), 1.0, jnp.float32)
    fn_full = lax.dot_general(mat, ones_c, (((0,), (0,)), ((), ())),
                              preferred_element_type=jnp.float32)
    fn_all = fn_full[0:C, :]
    valid = (fp_all != 0.0) & (fn_all != 0.0)
    precision = jnp.where(valid, tp / jnp.where(fp_all == 0.0, 1.0, fp_all), 0.0)
    recall = jnp.where(valid, tp / jnp.where(fn_all == 0.0, 1.0, fn_all), 0.0)
    pr_valid = (precision != 0.0) & (recall != 0.0)
    denom = jnp.where(pr_valid, precision + recall, 1.0)
    o_ref[...] = jnp.where(pr_valid, 2.0 * precision * recall / denom, 0.0)


def _dice_call(partials):
    flat = partials.reshape(NW * NROWS, ROW)
    return pl.pallas_call(
        _dice_body,
        out_shape=jax.ShapeDtypeStruct((C, 1), jnp.float32),
    )(flat)


def kernel(output, target):
    b, c, h, w = output.shape
    idx = _argmax_call(output, target.astype(jnp.int32), rows=128)
    partials = _sc_hist_call(idx)
    f2 = _dice_call(partials)
    return f2.reshape(C)


# speedup vs baseline: 1.0414x; 1.0414x over previous
import functools

import jax
import jax.numpy as jnp
from jax import lax
from jax.experimental import pallas as pl
from jax.experimental.pallas import tpu as pltpu
from jax.experimental.pallas import tpu_sc as plsc

C = 21
ROW = 32
NROWS = C + 1
HW = ROW * NROWS
DEAD = C * ROW

L = 16
NSC = 2
NSUB = 16
NW = NSC * NSUB


def _argmax_body(x_ref, t_ref, o_ref):
    x = x_ref[0]
    t = t_ref[0]
    m = x[0]
    seg = jnp.zeros(m.shape, jnp.int32)
    for c in range(1, C):
        xc = x[c]
        gt = xc > m
        m = jnp.where(gt, xc, m)
        seg = jnp.where(gt, c, seg)
    valid = (t >= 0) & (t < C)
    idx = jnp.where(valid, t * ROW + seg, DEAD)
    o_ref[...] = idx.reshape(o_ref.shape)


def _argmax_call(x4, t3, rows):
    b, _, h, w = x4.shape
    nb = h // rows
    return pl.pallas_call(
        _argmax_body,
        grid=(b, nb),
        in_specs=[
            pl.BlockSpec((1, C, rows, w), lambda i, j: (i, 0, j, 0)),
            pl.BlockSpec((1, rows, w), lambda i, j: (i, j, 0)),
        ],
        out_specs=pl.BlockSpec((rows * w,), lambda i, j, nb=nb: (i * nb + j,)),
        out_shape=jax.ShapeDtypeStruct((b * h * w,), jnp.int32),
    )(x4, t3)


HSTRIDE = HW + 1


NCHUNK = 4


def _sc_hist_body(idx_hbm, part_hbm, idx_v, hist_v, fold_v, sem0, sem1,
                  *, per_w):
    wid = lax.axis_index("s") * NSC + lax.axis_index("c")
    base = wid * per_w
    ch = per_w // NCHUNK
    sems = (sem0, sem1)

    copies = [
        pltpu.make_async_copy(idx_hbm.at[pl.ds(base + k * ch, ch)],
                              idx_v.at[k % 2], sems[k % 2])
        for k in range(NCHUNK)
    ]
    copies[0].start()

    zeros = jnp.zeros((L,), jnp.float32)
    lane_iota = lax.iota(jnp.int32, L)
    for i in range(HSTRIDE):
        plsc.store_scatter(hist_v, [jnp.minimum(i * L + lane_iota,
                                                L * HSTRIDE - 1)], zeros)

    lane_base = lane_iota * HSTRIDE
    ones = jnp.full((L,), 1.0, jnp.float32)

    for k in range(NCHUNK):
        copies[k].wait()
        if k + 1 < NCHUNK:
            copies[k + 1].start()

        @plsc.parallel_loop(0, ch, L, unroll=16)
        def _(i, k=k):
            v = idx_v[k % 2, pl.ds(i, L)]
            plsc.addupdate_scatter(hist_v, [lane_base + v], ones)

    for c in range(HW // L):
        acc = zeros
        for r in range(L):
            acc = acc + plsc.load_gather(hist_v, [r * HSTRIDE + c * L + lane_iota])
        fold_v[pl.ds(c * L, L)] = acc

    pltpu.sync_copy(fold_v, part_hbm.at[wid])


def _sc_hist_call(idx_flat):
    n = idx_flat.shape[0]
    per_w = n // NW
    mesh = plsc.VectorSubcoreMesh(core_axis_name="c", subcore_axis_name="s")
    return pl.kernel(
        functools.partial(_sc_hist_body, per_w=per_w),
        out_type=jax.ShapeDtypeStruct((NW, HW), jnp.float32),
        mesh=mesh,
        compiler_params=pltpu.CompilerParams(needs_layout_passes=False),
        scratch_types=[
            pltpu.VMEM((2, per_w // NCHUNK), jnp.int32),
            pltpu.VMEM((L * HSTRIDE,), jnp.float32),
            pltpu.VMEM((HW,), jnp.float32),
            pltpu.SemaphoreType.DMA,
            pltpu.SemaphoreType.DMA,
        ],
    )(idx_flat)


def _dice_body(p_ref, o_ref):
    p = p_ref[...]
    q = jnp.sum(p, axis=0, keepdims=True)
    kk = lax.broadcasted_iota(jnp.int32, (HW, ROW), 0)
    cc = lax.broadcasted_iota(jnp.int32, (HW, ROW), 1)
    tp_sel = (kk == (ROW + 1) * cc).astype(jnp.float32)
    fp_sel = (kk // ROW == cc).astype(jnp.float32)
    fn_sel = ((kk % ROW == cc) & (kk // ROW < C)).astype(jnp.float32)
    dims = (((1,), (0,)), ((), ()))
    tp = lax.dot_general(q, tp_sel, dims, preferred_element_type=jnp.float32)
    fp_all = lax.dot_general(q, fp_sel, dims, preferred_element_type=jnp.float32)
    fn_all = lax.dot_general(q, fn_sel, dims, preferred_element_type=jnp.float32)
    valid = (fp_all != 0.0) & (fn_all != 0.0)
    precision = jnp.where(valid, tp / jnp.where(fp_all == 0.0, 1.0, fp_all), 0.0)
    recall = jnp.where(valid, tp / jnp.where(fn_all == 0.0, 1.0, fn_all), 0.0)
    pr_valid = (precision != 0.0) & (recall != 0.0)
    denom = jnp.where(pr_valid, precision + recall, 1.0)
    f2 = jnp.where(pr_valid, 2.0 * precision * recall / denom, 0.0)
    o_ref[...] = f2[0, 0:C]


def _dice_call(partials):
    return pl.pallas_call(
        _dice_body,
        out_shape=jax.ShapeDtypeStruct((C,), jnp.float32),
    )(partials)


def kernel(output, target):
    b, c, h, w = output.shape
    idx = _argmax_call(output, target.astype(jnp.int32), rows=128)
    partials = _sc_hist_call(idx)
    return _dice_call(partials)
